# baseline (speedup 1.0000x reference)
import functools

import jax
import jax.numpy as jnp
from jax.experimental import pallas as pl
from jax.experimental.pallas import tpu as pltpu
from jax.experimental.pallas import tpu_sc as plsc

_N = 512
_D = 128
_B = 4
_KS = (20, 20, 5)
_BIG = 3.0e38
_NC, _NS, _L = 2, 16, 16
_NW = _NC * _NS


def _relu(x):
    return jnp.maximum(x, 0.0)


def _dot(a, b):
    return jnp.dot(a, b, preferred_element_type=jnp.float32)


def _head(P, W1t, b1, W2t, b2, Wgrt, Wgft, k, bid):
    f = _relu(_dot(P, W1t) + b1)
    f = _relu(_dot(f, W2t) + b2)
    x2 = jnp.sum(P * P, axis=1, keepdims=True)
    gram = jnp.zeros((_N, _N), jnp.float32)
    for c in range(3):
        col = P[:, c:c + 1]
        gram = gram + col * jnp.transpose(col)
    dist = x2 + jnp.transpose(x2) - 2.0 * gram
    u = _dot(f, Wgrt)
    v = _dot(f, Wgft - Wgrt)
    iota_i = jax.lax.broadcasted_iota(jnp.int32, (_N, 1), 0)
    iota2 = jnp.concatenate(
        [(iota_i // 256).astype(jnp.float32),
         (iota_i % 256).astype(jnp.float32)], axis=1)
    t = jnp.full((_N, 1), -_BIG, jnp.float32)
    cols = []
    for _ in range(k):
        cand = jnp.where(dist > t, dist, _BIG)
        m = jnp.min(cand, axis=1, keepdims=True)
        onehot = jnp.where(cand == m, 1.0, 0.0)
        d2 = _dot(onehot, iota2)
        cols.append(d2[:, 0:1] * 256.0 + d2[:, 1:2])
        t = m
    idx = (jnp.concatenate(cols, axis=1) + 0.5).astype(jnp.int32) + _N * bid
    return f, u, v, idx


def _tail(P, maxu, v, f, K, fvec, WhGt, WhFt, WhCt, WhKt, Wdt):
    g = _relu(maxu + v)
    Kc = _dot(g, WhGt) + _dot(f, WhFt) + _dot(fvec, WhCt)
    if K is not None:
        Kc = Kc + _dot(K, WhKt)
    Kc = _relu(Kc)
    return P + jnp.tanh(_dot(Kc, Wdt)), Kc


def _wspec(shape):
    return pl.BlockSpec(shape, lambda i: (0,) * len(shape))


_PSPEC = pl.BlockSpec((1, _N, 3), lambda i: (i, 0, 0))
_FSPEC = pl.BlockSpec((1, _N, _D), lambda i: (i, 0, 0))
_FEASPEC = pl.BlockSpec((1, 1, _D), lambda i: (i, 0, 0))
_CP = pltpu.CompilerParams(dimension_semantics=("parallel",))


def _stage0(seedT, W1t, b1, W2t, b2, Wgrt, Wgft):
    k = _KS[0]

    def body(seedT_ref, W1t_ref, b1_ref, W2t_ref, b2_ref, Wgr_ref, Wgf_ref,
             u_ref, idx_ref, v_ref, f_ref):
        bid = pl.program_id(0)
        f, u, v, idx = _head(seedT_ref[0], W1t_ref[0], b1_ref[0],
                             W2t_ref[0], b2_ref[0], Wgr_ref[0], Wgf_ref[0],
                             k, bid)
        u_ref[0], idx_ref[0], v_ref[0], f_ref[0] = u, idx, v, f

    return pl.pallas_call(
        body, grid=(_B,),
        in_specs=[_PSPEC, _wspec((1, 3, _D)), _wspec((1, 1, _D)),
                  _wspec((1, _D, _D)), _wspec((1, 1, _D)),
                  _wspec((1, _D, _D)), _wspec((1, _D, _D))],
        out_specs=[_FSPEC, pl.BlockSpec((1, _N, k), lambda i: (i, 0, 0)),
                   _FSPEC, _FSPEC],
        out_shape=[jax.ShapeDtypeStruct((_B, _N, _D), jnp.float32),
                   jax.ShapeDtypeStruct((_B, _N, k), jnp.int32),
                   jax.ShapeDtypeStruct((_B, _N, _D), jnp.float32),
                   jax.ShapeDtypeStruct((_B, _N, _D), jnp.float32)],
        compiler_params=_CP,
    )(seedT, W1t, b1, W2t, b2, Wgrt, Wgft)


def _stage_mid(lt, P, maxu, v, f, K, feaT, W1t, b1, W2t, b2, Wgrt, Wgft,
               WhGt, WhFt, WhCt, WhKt, Wdt):
    k = _KS[lt + 1]
    with_K = lt > 0

    def body(P_ref, maxu_ref, v_ref, f_ref, K_ref, fea_ref,
             W1t_ref, b1_ref, W2t_ref, b2_ref, Wgr_ref, Wgf_ref,
             WhG_ref, WhF_ref, WhC_ref, WhK_ref, Wdt_ref,
             o_ref, Kc_ref, u_ref, idx_ref, vo_ref, fo_ref):
        bid = pl.program_id(0)
        Kin = K_ref[0] if with_K else None
        Pn, Kc = _tail(P_ref[0], maxu_ref[0], v_ref[0], f_ref[0], Kin,
                       fea_ref[0], WhG_ref[0], WhF_ref[0], WhC_ref[0],
                       WhK_ref[0], Wdt_ref[0])
        o_ref[0], Kc_ref[0] = Pn, Kc
        fn, un, vn, idxn = _head(Pn, W1t_ref[0], b1_ref[0], W2t_ref[0],
                                 b2_ref[0], Wgr_ref[0], Wgf_ref[0], k, bid)
        u_ref[0], idx_ref[0], vo_ref[0], fo_ref[0] = un, idxn, vn, fn

    return pl.pallas_call(
        body, grid=(_B,),
        in_specs=[_PSPEC, _FSPEC, _FSPEC, _FSPEC, _FSPEC, _FEASPEC,
                  _wspec((1, 3, _D)), _wspec((1, 1, _D)),
                  _wspec((1, _D, _D)), _wspec((1, 1, _D)),
                  _wspec((1, _D, _D)), _wspec((1, _D, _D)),
                  _wspec((1, _D, _D)), _wspec((1, _D, _D)),
                  _wspec((1, _D, _D)), _wspec((1, _D, _D)),
                  _wspec((1, _D, 3))],
        out_specs=[_PSPEC, _FSPEC, _FSPEC,
                   pl.BlockSpec((1, _N, k), lambda i: (i, 0, 0)),
                   _FSPEC, _FSPEC],
        out_shape=[jax.ShapeDtypeStruct((_B, _N, 3), jnp.float32),
                   jax.ShapeDtypeStruct((_B, _N, _D), jnp.float32),
                   jax.ShapeDtypeStruct((_B, _N, _D), jnp.float32),
                   jax.ShapeDtypeStruct((_B, _N, k), jnp.int32),
                   jax.ShapeDtypeStruct((_B, _N, _D), jnp.float32),
                   jax.ShapeDtypeStruct((_B, _N, _D), jnp.float32)],
        compiler_params=_CP,
    )(P, maxu, v, f, K, feaT, W1t, b1, W2t, b2, Wgrt, Wgft,
      WhGt, WhFt, WhCt, WhKt, Wdt)


def _stage_last(P, maxu, v, f, K, feaT, WhGt, WhFt, WhCt, WhKt, Wdt):
    def body(P_ref, maxu_ref, v_ref, f_ref, K_ref, fea_ref,
             WhG_ref, WhF_ref, WhC_ref, WhK_ref, Wdt_ref, o_ref):
        Pn, _ = _tail(P_ref[0], maxu_ref[0], v_ref[0], f_ref[0], K_ref[0],
                      fea_ref[0], WhG_ref[0], WhF_ref[0], WhC_ref[0],
                      WhK_ref[0], Wdt_ref[0])
        o_ref[0] = Pn

    return pl.pallas_call(
        body, grid=(_B,),
        in_specs=[_PSPEC, _FSPEC, _FSPEC, _FSPEC, _FSPEC, _FEASPEC,
                  _wspec((1, _D, _D)), _wspec((1, _D, _D)),
                  _wspec((1, _D, _D)), _wspec((1, _D, _D)),
                  _wspec((1, _D, 3))],
        out_specs=[_PSPEC],
        out_shape=[jax.ShapeDtypeStruct((_B, _N, 3), jnp.float32)],
        compiler_params=_CP,
    )(P, maxu, v, f, K, feaT, WhGt, WhFt, WhCt, WhKt, Wdt)[0]


@functools.cache
def _sc_gather_max(k, sub):
    pts = _B * _N
    pp = pts // _NW
    nch = pp // sub

    mesh = plsc.VectorSubcoreMesh(core_axis_name="c", subcore_axis_name="s")

    @functools.partial(
        pl.kernel, mesh=mesh,
        out_type=jax.ShapeDtypeStruct((pts, _D), jnp.float32),
        scratch_types=[
            pltpu.VMEM((pp * k,), jnp.int32),
            pltpu.VMEM((sub * k, _D), jnp.float32),
            pltpu.VMEM((sub, _D), jnp.float32),
            pltpu.SemaphoreType.DMA,
        ],
    )
    def sc_k(u_hbm, idx_hbm, out_hbm, idx_v, rows_v, out_v, sem):
        wid = jax.lax.axis_index("s") * _NC + jax.lax.axis_index("c")
        base = wid * pp
        pltpu.sync_copy(idx_hbm.at[pl.ds(pl.multiple_of(base * k, 8), pp * k)],
                        idx_v)

        def chunk(h, carry):
            off = pl.multiple_of(h * (sub * k), 8)
            pltpu.async_copy(u_hbm.at[idx_v.at[pl.ds(off, sub * k)]],
                             rows_v, sem).wait()
            for p in range(sub):
                for c in range(_D // _L):
                    acc = rows_v[p * k, pl.ds(c * _L, _L)]
                    for j in range(1, k):
                        acc = jnp.maximum(
                            acc, rows_v[p * k + j, pl.ds(c * _L, _L)])
                    out_v[p, pl.ds(c * _L, _L)] = acc
            pltpu.sync_copy(out_v, out_hbm.at[pl.ds(base + h * sub, sub)])
            return carry

        jax.lax.fori_loop(0, nch, chunk, 0)

    return sc_k


def _gather_max(u, idx, k, sub):
    uf = jnp.reshape(u, (_B * _N, _D))
    idxf = jnp.reshape(idx, (_B * _N * k,))
    out = _sc_gather_max(k, sub)(uf, idxf)
    return jnp.reshape(out, (_B, _N, _D))


def kernel(seed, fea, W1, b1, W2, b2, Wg, Wh, Wd):
    seedT = jnp.transpose(seed, (0, 2, 1))
    feaT = jnp.transpose(fea, (0, 2, 1))
    t = lambda w: jnp.transpose(w, (0, 2, 1))
    W1t, W2t, Wdt = t(W1), t(W2), t(Wd)
    b1r, b2r = b1[:, None, :], b2[:, None, :]
    Wgrt, Wgft = t(Wg[:, :, :_D]), t(Wg[:, :, _D:])
    WhGt, WhFt = t(Wh[:, :, 0:_D]), t(Wh[:, :, _D:2 * _D])
    WhCt, WhKt = t(Wh[:, :, 2 * _D:3 * _D]), t(Wh[:, :, 3 * _D:])
    w = lambda a, l: a[l:l + 1]

    u0, idx0, v0, f0 = _stage0(
        seedT, w(W1t, 0), w(b1r, 0), w(W2t, 0), w(b2r, 0),
        w(Wgrt, 0), w(Wgft, 0))
    maxu0 = _gather_max(u0, idx0, _KS[0], 4)
    o1, K0, u1, idx1, v1, f1 = _stage_mid(
        0, seedT, maxu0, v0, f0, v0, feaT,
        w(W1t, 1), w(b1r, 1), w(W2t, 1), w(b2r, 1), w(Wgrt, 1), w(Wgft, 1),
        w(WhGt, 0), w(WhFt, 0), w(WhCt, 0), w(WhKt, 0), w(Wdt, 0))
    maxu1 = _gather_max(u1, idx1, _KS[1], 4)
    o2, K1, u2, idx2, v2, f2 = _stage_mid(
        1, o1, maxu1, v1, f1, K0, feaT,
        w(W1t, 2), w(b1r, 2), w(W2t, 2), w(b2r, 2), w(Wgrt, 2), w(Wgft, 2),
        w(WhGt, 1), w(WhFt, 1), w(WhCt, 1), w(WhKt, 1), w(Wdt, 1))
    maxu2 = _gather_max(u2, idx2, _KS[2], 8)
    o3 = _stage_last(
        o2, maxu2, v2, f2, K1, feaT,
        w(WhGt, 2), w(WhFt, 2), w(WhCt, 2), w(WhKt, 2), w(Wdt, 2))
    return (seedT, o1, jnp.repeat(o2, 4, axis=1), jnp.repeat(o3, 16, axis=1))


# speedup vs baseline: 2.5712x; 2.5712x over previous
import jax
import jax.numpy as jnp
from jax.experimental import pallas as pl
from jax.experimental.pallas import tpu as pltpu

_N = 512
_D = 128
_KS = (20, 20, 5)
_BIG = 3.0e38
_JC = 32


def _topk_gathermax(dist, u, k):
    def round_fn(_, carry):
        t, macc = carry
        cand = jnp.where(dist > t, dist, _BIG)
        m = jnp.min(cand, axis=1, keepdims=True)
        onehot = jnp.where(cand == m, 1.0, 0.0)
        picked = jnp.dot(onehot, u, preferred_element_type=jnp.float32)
        return m, jnp.maximum(macc, picked)

    _, macc = jax.lax.fori_loop(
        0, k, round_fn,
        (jnp.full((_N, 1), -_BIG, jnp.float32),
         jnp.full((_N, _D), -_BIG, jnp.float32)),
        unroll=4)
    return macc


def _body(seedT_ref, feaT_ref, W1t_ref, b1_ref, W2t_ref, b2_ref,
          Wgrt_ref, Wgft_ref, WhGt_ref, WhFt_ref, WhCt_ref, WhKt_ref,
          Wdt_ref, o1_ref, o2_ref, o3_ref):
    dot = lambda a, b: jnp.dot(a, b, preferred_element_type=jnp.float32)
    P = seedT_ref[0]
    fvec = feaT_ref[0]
    K = jnp.zeros((_N, _D), jnp.float32)
    outs = (o1_ref, o2_ref, o3_ref)

    for l in range(3):
        f = jnp.maximum(dot(P, W1t_ref[l]) + b1_ref[l], 0.0)
        f = jnp.maximum(dot(f, W2t_ref[l]) + b2_ref[l], 0.0)
        x2 = jnp.sum(P * P, axis=1, keepdims=True)
        gram = jnp.zeros((_N, _N), jnp.float32)
        for c in range(3):
            col = P[:, c:c + 1]
            gram = gram + col * jnp.transpose(col)
        dist = x2 + jnp.transpose(x2) - 2.0 * gram
        u = dot(f, Wgrt_ref[l])
        v = dot(f, Wgft_ref[l] - Wgrt_ref[l])
        g = jnp.maximum(_topk_gathermax(dist, u, _KS[l]) + v, 0.0)
        Kc = jnp.maximum(
            dot(g, WhGt_ref[l]) + dot(f, WhFt_ref[l])
            + dot(fvec, WhCt_ref[l]) + dot(K, WhKt_ref[l]), 0.0)
        P = P + jnp.tanh(dot(Kc, Wdt_ref[l]))
        outs[l][0] = P
        K = Kc


def _call(seedT, feaT, W1t, b1r, W2t, b2r, Wgrt, Wgft,
          WhGt, WhFt, WhCt, WhKt, Wdt, interpret=False):
    b = seedT.shape[0]
    wspec = lambda shape: pl.BlockSpec(shape, lambda i: (0,) * len(shape))
    out_shape = [jax.ShapeDtypeStruct((b, _N, 3), jnp.float32)] * 3
    out_spec = pl.BlockSpec((1, _N, 3), lambda i: (i, 0, 0))
    return pl.pallas_call(
        _body,
        grid=(b,),
        in_specs=[
            pl.BlockSpec((1, _N, 3), lambda i: (i, 0, 0)),
            pl.BlockSpec((1, 1, _D), lambda i: (i, 0, 0)),
            wspec((3, 3, _D)),
            wspec((3, 1, _D)),
            wspec((3, _D, _D)),
            wspec((3, 1, _D)),
            wspec((3, _D, _D)),
            wspec((3, _D, _D)),
            wspec((3, _D, _D)),
            wspec((3, _D, _D)),
            wspec((3, _D, _D)),
            wspec((3, _D, _D)),
            wspec((3, _D, 3)),
        ],
        out_specs=[out_spec] * 3,
        out_shape=out_shape,
        compiler_params=pltpu.CompilerParams(
            dimension_semantics=("parallel",)),
        interpret=interpret,
    )(seedT, feaT, W1t, b1r, W2t, b2r, Wgrt, Wgft, WhGt, WhFt, WhCt,
      WhKt, Wdt)


def kernel(seed, fea, W1, b1, W2, b2, Wg, Wh, Wd):
    seedT = jnp.transpose(seed, (0, 2, 1))
    feaT = jnp.transpose(fea, (0, 2, 1))
    t = lambda w: jnp.transpose(w, (0, 2, 1))
    o1, o2, o3 = _call(
        seedT, feaT,
        t(W1), b1[:, None, :], t(W2), b2[:, None, :],
        t(Wg[:, :, :_D]), t(Wg[:, :, _D:]),
        t(Wh[:, :, 0:_D]), t(Wh[:, :, _D:2 * _D]),
        t(Wh[:, :, 2 * _D:3 * _D]), t(Wh[:, :, 3 * _D:]),
        t(Wd))
    pred2 = jnp.repeat(o2, 4, axis=1)
    pred3 = jnp.repeat(o3, 16, axis=1)
    return (seedT, o1, pred2, pred3)


# speedup vs baseline: 2.6513x; 1.0312x over previous
import jax
import jax.numpy as jnp
from jax.experimental import pallas as pl
from jax.experimental.pallas import tpu as pltpu

_N = 512
_D = 128
_KS = (20, 20, 5)
_BIG = 3.0e38
_JC = 32


def _topk_gathermax(dist, u, k):
    def round_fn(_, carry):
        t, macc = carry
        cand = jnp.where(dist > t, dist, _BIG)
        m = jnp.min(cand, axis=1, keepdims=True)
        onehot = jnp.where(cand == m, 1.0, 0.0)
        picked = jnp.dot(onehot, u, preferred_element_type=jnp.float32)
        return m, jnp.maximum(macc, picked)

    _, macc = jax.lax.fori_loop(
        0, k, round_fn,
        (jnp.full((_N, 1), -_BIG, jnp.float32),
         jnp.full((_N, _D), -_BIG, jnp.float32)),
        unroll=5)
    return macc


def _body(seedT_ref, feaT_ref, W1t_ref, b1_ref, W2t_ref, b2_ref,
          Wgrt_ref, Wgft_ref, WhGt_ref, WhFt_ref, WhCt_ref, WhKt_ref,
          Wdt_ref, o1_ref, o2_ref, o3_ref):
    dot = lambda a, b: jnp.dot(a, b, preferred_element_type=jnp.float32)
    P = seedT_ref[0]
    fvec = feaT_ref[0]
    K = jnp.zeros((_N, _D), jnp.float32)
    outs = (o1_ref, o2_ref, o3_ref)

    for l in range(3):
        f = jnp.maximum(dot(P, W1t_ref[l]) + b1_ref[l], 0.0)
        f = jnp.maximum(dot(f, W2t_ref[l]) + b2_ref[l], 0.0)
        x2 = jnp.sum(P * P, axis=1, keepdims=True)
        gram = jnp.zeros((_N, _N), jnp.float32)
        for c in range(3):
            col = P[:, c:c + 1]
            gram = gram + col * jnp.transpose(col)
        dist = x2 + jnp.transpose(x2) - 2.0 * gram
        u = dot(f, Wgrt_ref[l])
        v = dot(f, Wgft_ref[l] - Wgrt_ref[l])
        g = jnp.maximum(_topk_gathermax(dist, u, _KS[l]) + v, 0.0)
        Kc = jnp.maximum(
            dot(g, WhGt_ref[l]) + dot(f, WhFt_ref[l])
            + dot(fvec, WhCt_ref[l]) + dot(K, WhKt_ref[l]), 0.0)
        P = P + jnp.tanh(dot(Kc, Wdt_ref[l]))
        outs[l][0] = P
        K = Kc


def _call(seedT, feaT, W1t, b1r, W2t, b2r, Wgrt, Wgft,
          WhGt, WhFt, WhCt, WhKt, Wdt, interpret=False):
    b = seedT.shape[0]
    wspec = lambda shape: pl.BlockSpec(shape, lambda i: (0,) * len(shape))
    out_shape = [jax.ShapeDtypeStruct((b, _N, 3), jnp.float32)] * 3
    out_spec = pl.BlockSpec((1, _N, 3), lambda i: (i, 0, 0))
    return pl.pallas_call(
        _body,
        grid=(b,),
        in_specs=[
            pl.BlockSpec((1, _N, 3), lambda i: (i, 0, 0)),
            pl.BlockSpec((1, 1, _D), lambda i: (i, 0, 0)),
            wspec((3, 3, _D)),
            wspec((3, 1, _D)),
            wspec((3, _D, _D)),
            wspec((3, 1, _D)),
            wspec((3, _D, _D)),
            wspec((3, _D, _D)),
            wspec((3, _D, _D)),
            wspec((3, _D, _D)),
            wspec((3, _D, _D)),
            wspec((3, _D, _D)),
            wspec((3, _D, 3)),
        ],
        out_specs=[out_spec] * 3,
        out_shape=out_shape,
        compiler_params=pltpu.CompilerParams(
            dimension_semantics=("parallel",)),
        interpret=interpret,
    )(seedT, feaT, W1t, b1r, W2t, b2r, Wgrt, Wgft, WhGt, WhFt, WhCt,
      WhKt, Wdt)


def kernel(seed, fea, W1, b1, W2, b2, Wg, Wh, Wd):
    seedT = jnp.transpose(seed, (0, 2, 1))
    feaT = jnp.transpose(fea, (0, 2, 1))
    t = lambda w: jnp.transpose(w, (0, 2, 1))
    o1, o2, o3 = _call(
        seedT, feaT,
        t(W1), b1[:, None, :], t(W2), b2[:, None, :],
        t(Wg[:, :, :_D]), t(Wg[:, :, _D:]),
        t(Wh[:, :, 0:_D]), t(Wh[:, :, _D:2 * _D]),
        t(Wh[:, :, 2 * _D:3 * _D]), t(Wh[:, :, 3 * _D:]),
        t(Wd))
    pred2 = jnp.repeat(o2, 4, axis=1)
    pred3 = jnp.repeat(o3, 16, axis=1)
    return (seedT, o1, pred2, pred3)


# speedup vs baseline: 2.7161x; 1.0244x over previous
import jax
import jax.numpy as jnp
from jax.experimental import pallas as pl
from jax.experimental.pallas import tpu as pltpu

_N = 512
_D = 128
_KS = (20, 20, 5)
_BIG = 3.0e38
_JC = 32


def _topk_gathermax(dist, u, k):
    def round_fn(_, carry):
        t, macc = carry
        cand = jnp.where(dist > t, dist, _BIG)
        m = jnp.min(cand, axis=1, keepdims=True)
        onehot = jnp.where(cand == m, 1.0, 0.0)
        picked = jnp.dot(onehot, u, preferred_element_type=jnp.float32)
        return m, jnp.maximum(macc, picked)

    _, macc = jax.lax.fori_loop(
        0, k, round_fn,
        (jnp.full((_N, 1), -_BIG, jnp.float32),
         jnp.full((_N, _D), -_BIG, jnp.float32)),
        unroll=10)
    return macc


def _body(seedT_ref, feaT_ref, W1t_ref, b1_ref, W2t_ref, b2_ref,
          Wgrt_ref, Wgft_ref, WhGt_ref, WhFt_ref, WhCt_ref, WhKt_ref,
          Wdt_ref, o1_ref, o2_ref, o3_ref):
    dot = lambda a, b: jnp.dot(a, b, preferred_element_type=jnp.float32)
    P = seedT_ref[0]
    fvec = feaT_ref[0]
    K = jnp.zeros((_N, _D), jnp.float32)
    outs = (o1_ref, o2_ref, o3_ref)

    for l in range(3):
        f = jnp.maximum(dot(P, W1t_ref[l]) + b1_ref[l], 0.0)
        f = jnp.maximum(dot(f, W2t_ref[l]) + b2_ref[l], 0.0)
        x2 = jnp.sum(P * P, axis=1, keepdims=True)
        gram = jnp.zeros((_N, _N), jnp.float32)
        for c in range(3):
            col = P[:, c:c + 1]
            gram = gram + col * jnp.transpose(col)
        dist = x2 + jnp.transpose(x2) - 2.0 * gram
        u = dot(f, Wgrt_ref[l])
        v = dot(f, Wgft_ref[l] - Wgrt_ref[l])
        g = jnp.maximum(_topk_gathermax(dist, u, _KS[l]) + v, 0.0)
        Kc = jnp.maximum(
            dot(g, WhGt_ref[l]) + dot(f, WhFt_ref[l])
            + dot(fvec, WhCt_ref[l]) + dot(K, WhKt_ref[l]), 0.0)
        P = P + jnp.tanh(dot(Kc, Wdt_ref[l]))
        outs[l][0] = P
        K = Kc


def _call(seedT, feaT, W1t, b1r, W2t, b2r, Wgrt, Wgft,
          WhGt, WhFt, WhCt, WhKt, Wdt, interpret=False):
    b = seedT.shape[0]
    wspec = lambda shape: pl.BlockSpec(shape, lambda i: (0,) * len(shape))
    out_shape = [jax.ShapeDtypeStruct((b, _N, 3), jnp.float32)] * 3
    out_spec = pl.BlockSpec((1, _N, 3), lambda i: (i, 0, 0))
    return pl.pallas_call(
        _body,
        grid=(b,),
        in_specs=[
            pl.BlockSpec((1, _N, 3), lambda i: (i, 0, 0)),
            pl.BlockSpec((1, 1, _D), lambda i: (i, 0, 0)),
            wspec((3, 3, _D)),
            wspec((3, 1, _D)),
            wspec((3, _D, _D)),
            wspec((3, 1, _D)),
            wspec((3, _D, _D)),
            wspec((3, _D, _D)),
            wspec((3, _D, _D)),
            wspec((3, _D, _D)),
            wspec((3, _D, _D)),
            wspec((3, _D, _D)),
            wspec((3, _D, 3)),
        ],
        out_specs=[out_spec] * 3,
        out_shape=out_shape,
        compiler_params=pltpu.CompilerParams(
            dimension_semantics=("parallel",)),
        interpret=interpret,
    )(seedT, feaT, W1t, b1r, W2t, b2r, Wgrt, Wgft, WhGt, WhFt, WhCt,
      WhKt, Wdt)


def kernel(seed, fea, W1, b1, W2, b2, Wg, Wh, Wd):
    seedT = jnp.transpose(seed, (0, 2, 1))
    feaT = jnp.transpose(fea, (0, 2, 1))
    t = lambda w: jnp.transpose(w, (0, 2, 1))
    o1, o2, o3 = _call(
        seedT, feaT,
        t(W1), b1[:, None, :], t(W2), b2[:, None, :],
        t(Wg[:, :, :_D]), t(Wg[:, :, _D:]),
        t(Wh[:, :, 0:_D]), t(Wh[:, :, _D:2 * _D]),
        t(Wh[:, :, 2 * _D:3 * _D]), t(Wh[:, :, 3 * _D:]),
        t(Wd))
    pred2 = jnp.repeat(o2, 4, axis=1)
    pred3 = jnp.repeat(o3, 16, axis=1)
    return (seedT, o1, pred2, pred3)


# speedup vs baseline: 2.8026x; 1.0318x over previous
import jax
import jax.numpy as jnp
from jax.experimental import pallas as pl
from jax.experimental.pallas import tpu as pltpu

_N = 512
_D = 128
_KS = (20, 20, 5)
_BIG = 3.0e38
_JC = 32


def _topk_gathermax(dist, u, k):
    def round_fn(_, carry):
        t, macc = carry
        cand = jnp.where(dist > t, dist, _BIG)
        m = jnp.min(cand, axis=1, keepdims=True)
        onehot = jnp.where(cand == m, 1.0, 0.0)
        picked = jnp.dot(onehot, u, preferred_element_type=jnp.float32)
        return m, jnp.maximum(macc, picked)

    _, macc = jax.lax.fori_loop(
        0, k, round_fn,
        (jnp.full((_N, 1), -_BIG, jnp.float32),
         jnp.full((_N, _D), -_BIG, jnp.float32)),
        unroll=k)
    return macc


def _body(seedT_ref, feaT_ref, W1t_ref, b1_ref, W2t_ref, b2_ref,
          Wgrt_ref, Wgft_ref, WhGt_ref, WhFt_ref, WhCt_ref, WhKt_ref,
          Wdt_ref, o1_ref, o2_ref, o3_ref):
    dot = lambda a, b: jnp.dot(a, b, preferred_element_type=jnp.float32)
    P = seedT_ref[0]
    fvec = feaT_ref[0]
    K = jnp.zeros((_N, _D), jnp.float32)
    outs = (o1_ref, o2_ref, o3_ref)

    for l in range(3):
        f = jnp.maximum(dot(P, W1t_ref[l]) + b1_ref[l], 0.0)
        f = jnp.maximum(dot(f, W2t_ref[l]) + b2_ref[l], 0.0)
        x2 = jnp.sum(P * P, axis=1, keepdims=True)
        gram = jnp.zeros((_N, _N), jnp.float32)
        for c in range(3):
            col = P[:, c:c + 1]
            gram = gram + col * jnp.transpose(col)
        dist = x2 + jnp.transpose(x2) - 2.0 * gram
        u = dot(f, Wgrt_ref[l])
        v = dot(f, Wgft_ref[l] - Wgrt_ref[l])
        g = jnp.maximum(_topk_gathermax(dist, u, _KS[l]) + v, 0.0)
        Kc = jnp.maximum(
            dot(g, WhGt_ref[l]) + dot(f, WhFt_ref[l])
            + dot(fvec, WhCt_ref[l]) + dot(K, WhKt_ref[l]), 0.0)
        P = P + jnp.tanh(dot(Kc, Wdt_ref[l]))
        outs[l][0] = P
        K = Kc


def _call(seedT, feaT, W1t, b1r, W2t, b2r, Wgrt, Wgft,
          WhGt, WhFt, WhCt, WhKt, Wdt, interpret=False):
    b = seedT.shape[0]
    wspec = lambda shape: pl.BlockSpec(shape, lambda i: (0,) * len(shape))
    out_shape = [jax.ShapeDtypeStruct((b, _N, 3), jnp.float32)] * 3
    out_spec = pl.BlockSpec((1, _N, 3), lambda i: (i, 0, 0))
    return pl.pallas_call(
        _body,
        grid=(b,),
        in_specs=[
            pl.BlockSpec((1, _N, 3), lambda i: (i, 0, 0)),
            pl.BlockSpec((1, 1, _D), lambda i: (i, 0, 0)),
            wspec((3, 3, _D)),
            wspec((3, 1, _D)),
            wspec((3, _D, _D)),
            wspec((3, 1, _D)),
            wspec((3, _D, _D)),
            wspec((3, _D, _D)),
            wspec((3, _D, _D)),
            wspec((3, _D, _D)),
            wspec((3, _D, _D)),
            wspec((3, _D, _D)),
            wspec((3, _D, 3)),
        ],
        out_specs=[out_spec] * 3,
        out_shape=out_shape,
        compiler_params=pltpu.CompilerParams(
            dimension_semantics=("parallel",)),
        interpret=interpret,
    )(seedT, feaT, W1t, b1r, W2t, b2r, Wgrt, Wgft, WhGt, WhFt, WhCt,
      WhKt, Wdt)


def kernel(seed, fea, W1, b1, W2, b2, Wg, Wh, Wd):
    seedT = jnp.transpose(seed, (0, 2, 1))
    feaT = jnp.transpose(fea, (0, 2, 1))
    t = lambda w: jnp.transpose(w, (0, 2, 1))
    o1, o2, o3 = _call(
        seedT, feaT,
        t(W1), b1[:, None, :], t(W2), b2[:, None, :],
        t(Wg[:, :, :_D]), t(Wg[:, :, _D:]),
        t(Wh[:, :, 0:_D]), t(Wh[:, :, _D:2 * _D]),
        t(Wh[:, :, 2 * _D:3 * _D]), t(Wh[:, :, 3 * _D:]),
        t(Wd))
    pred2 = jnp.repeat(o2, 4, axis=1)
    pred3 = jnp.repeat(o3, 16, axis=1)
    return (seedT, o1, pred2, pred3)
